# Initial kernel scaffold; baseline (speedup 1.0000x reference)
import jax
import jax.numpy as jnp
from jax.experimental import pallas as pl


def kernel(x, core0, core1, core2):
    raise NotImplementedError("write your pallas kernel here")


# speedup vs baseline: 7.5714x; 7.5714x over previous
import functools

import jax
import jax.numpy as jnp
from jax import lax
from jax.experimental import pallas as pl
from jax.experimental.pallas import tpu as pltpu
from jax.experimental.pallas import tpu_sc as plsc

M0 = M1 = M2 = 100
Q0 = Q1 = Q2 = 4
R1 = 8
R2 = 8
NTOK = 16384 * 50
OUT_D = Q0 * Q1 * Q2

NC, NS, L = 2, 16, 16
NW = NC * NS
TPW = NTOK // NW
CHUNK = 128
NCHUNK = TPW // CHUNK


def _mm_body(a_ref, b_ref, o_ref):
    o_ref[...] = jnp.dot(a_ref[...], b_ref[...],
                         preferred_element_type=jnp.float32)


def _build_t01(core0, core1):
    a0 = core0.reshape(M0 * Q0, R1)
    w1 = core1.reshape(R1, M1 * Q1 * R2)
    m = pl.pallas_call(
        _mm_body,
        out_shape=jax.ShapeDtypeStruct((M0 * Q0, M1 * Q1 * R2), jnp.float32),
    )(a0, w1)
    t01 = m.reshape(M0, Q0, M1, Q1, R2).transpose(0, 2, 1, 3, 4)
    return t01.reshape(M0 * M1, Q0 * Q1 * R2)


def _sc_body(flat_hbm, t01_hbm, c2_hbm, out_hbm,
             x_v, idx_v, rows_v, c2_v, out_v, sem):
    wid = lax.axis_index("s") * NC + lax.axis_index("c")
    base = wid * TPW
    pltpu.sync_copy(c2_hbm, c2_v)

    def chunk_body(g, carry):
        tok0 = base + g * CHUNK
        pltpu.sync_copy(flat_hbm.at[pl.ds(tok0, CHUNK)], x_v)

        def idx_body(t, c):
            xv = x_v[pl.ds(t * L, L)]
            idx_v[pl.ds(t * L, L)] = lax.div(xv, 100)
            return c

        lax.fori_loop(0, CHUNK // L, idx_body, 0)
        pltpu.async_copy(t01_hbm.at[idx_v], rows_v, sem).wait()

        def t16_body(t, c):
            tokv = t * L + lax.iota(jnp.int32, L)
            xv = x_v[pl.ds(t * L, L)]
            i01 = idx_v[pl.ds(t * L, L)]
            i2 = xv - i01 * 100
            c2base = i2 * (R2 * Q2)
            c2g = [plsc.load_gather(c2_v, [c2base + k])
                   for k in range(R2 * Q2)]
            outbase = tokv * OUT_D
            for qq in range(Q0 * Q1):
                rg = [plsc.load_gather(
                          rows_v,
                          [tokv, jnp.full((L,), qq * R2 + r, jnp.int32)])
                      for r in range(R2)]
                for q2 in range(Q2):
                    acc = rg[0] * c2g[q2]
                    for r in range(1, R2):
                        acc = acc + rg[r] * c2g[r * Q2 + q2]
                    plsc.store_scatter(out_v, [outbase + (qq * Q2 + q2)], acc)
            return c

        lax.fori_loop(0, CHUNK // L, t16_body, 0)
        pltpu.sync_copy(out_v,
                        out_hbm.at[pl.ds(tok0 * OUT_D, CHUNK * OUT_D)])
        return carry

    lax.fori_loop(0, NCHUNK, chunk_body, 0)


_sc_kernel = functools.partial(
    pl.kernel,
    out_type=jax.ShapeDtypeStruct((NTOK * OUT_D,), jnp.float32),
    mesh=plsc.VectorSubcoreMesh(core_axis_name="c", subcore_axis_name="s"),
    scratch_types=[
        pltpu.VMEM((CHUNK,), jnp.int32),
        pltpu.VMEM((CHUNK,), jnp.int32),
        pltpu.VMEM((CHUNK, Q0 * Q1 * R2), jnp.float32),
        pltpu.VMEM((M2 * R2 * Q2,), jnp.float32),
        pltpu.VMEM((CHUNK * OUT_D,), jnp.float32),
        pltpu.SemaphoreType.DMA,
    ],
    compiler_params=pltpu.CompilerParams(needs_layout_passes=False),
)(_sc_body)


def kernel(x, core0, core1, core2):
    batch, sent = x.shape
    flat = x.reshape(-1).astype(jnp.int32)
    t01 = _build_t01(core0, core1)
    c2t = core2.reshape(R2, M2, Q2).transpose(1, 0, 2).reshape(M2 * R2 * Q2)
    out = _sc_kernel(flat, t01, c2t)
    return out.reshape(batch, sent, OUT_D)


# speedup vs baseline: 8.5015x; 1.1228x over previous
import functools

import jax
import jax.numpy as jnp
from jax import lax
from jax.experimental import pallas as pl
from jax.experimental.pallas import tpu as pltpu
from jax.experimental.pallas import tpu_sc as plsc

M0 = M1 = M2 = 100
Q0 = Q1 = Q2 = 4
R1 = 8
R2 = 8
NTOK = 16384 * 50
OUT_D = Q0 * Q1 * Q2
ROW_D = Q0 * Q1 * R2

NC, NS, L = 2, 16, 16
NW = NC * NS
TPW = NTOK // NW
CHUNK = 128
NCHUNK = TPW // CHUNK


def _mm_body(a_ref, b_ref, o_ref):
    o_ref[...] = jnp.dot(a_ref[...], b_ref[...],
                         preferred_element_type=jnp.float32)


def _build_t01(core0, core1):
    a0 = core0.reshape(M0 * Q0, R1)
    w1 = core1.reshape(R1, M1 * Q1 * R2)
    m = pl.pallas_call(
        _mm_body,
        out_shape=jax.ShapeDtypeStruct((M0 * Q0, M1 * Q1 * R2), jnp.float32),
    )(a0, w1)
    t01 = m.reshape(M0, Q0, M1, Q1, R2).transpose(0, 2, 1, 3, 4)
    return t01.reshape(M0 * M1, ROW_D)


def _sc_body(flat_hbm, t01_hbm, c2_hbm, out_hbm,
             x_v, idx_v, c2_v, rows_a, rows_b, out_a, out_b,
             gs_a, gs_b, os_a, os_b):
    wid = lax.axis_index("s") * NC + lax.axis_index("c")
    base = wid * TPW
    pltpu.sync_copy(c2_hbm, c2_v)
    pltpu.sync_copy(flat_hbm.at[pl.ds(base, TPW)], x_v)

    @plsc.parallel_loop(0, TPW // L)
    def _(t):
        xv = x_v[pl.ds(t * L, L)]
        idx_v[pl.ds(t * L, L)] = lax.div(xv, 100)

    def start_gather(c, rows_ref, sem):
        pltpu.async_copy(
            t01_hbm.at[idx_v.at[pl.ds(c * CHUNK, CHUNK)]], rows_ref, sem)

    def compute_chunk(c, rows_ref, out_ref, osem):
        @plsc.parallel_loop(0, CHUNK // L)
        def _(t):
            tokv = t * L + lax.iota(jnp.int32, L)
            off = c * CHUNK + t * L
            xv = x_v[pl.ds(off, L)]
            i01 = idx_v[pl.ds(off, L)]
            i2 = xv - i01 * 100
            c2base = i2 * (R2 * Q2)
            c2g = [plsc.load_gather(c2_v, [c2base + k])
                   for k in range(R2 * Q2)]
            outbase = tokv * OUT_D
            for qq in range(Q0 * Q1):
                rg = [plsc.load_gather(
                          rows_ref,
                          [tokv, jnp.full((L,), qq * R2 + r, jnp.int32)])
                      for r in range(R2)]
                for q2 in range(Q2):
                    acc = rg[0] * c2g[q2]
                    for r in range(1, R2):
                        acc = acc + rg[r] * c2g[r * Q2 + q2]
                    plsc.store_scatter(out_ref,
                                       [outbase + (qq * Q2 + q2)], acc)

        pltpu.async_copy(
            out_ref,
            out_hbm.at[pl.ds((base + c * CHUNK) * OUT_D, CHUNK * OUT_D)],
            osem)

    def wait_gather(c, rows_ref, sem):
        pltpu.make_async_copy(
            t01_hbm.at[idx_v.at[pl.ds(c * CHUNK, CHUNK)]], rows_ref, sem
        ).wait()

    def wait_out(c, out_ref, osem):
        pltpu.make_async_copy(
            out_ref,
            out_hbm.at[pl.ds((base + c * CHUNK) * OUT_D, CHUNK * OUT_D)],
            osem).wait()

    start_gather(0, rows_a, gs_a)

    bufs = ((rows_a, out_a, gs_a, os_a), (rows_b, out_b, gs_b, os_b))

    def pair_body(g2, carry):
        for b in range(2):
            rv, ov, gs, os = bufs[b]
            rn, _, gn, _ = bufs[1 - b]
            c = g2 * 2 + b

            @pl.when(c + 1 < NCHUNK)
            def _():
                start_gather(c + 1, rn, gn)

            wait_gather(c, rv, gs)

            @pl.when(c >= 2)
            def _():
                wait_out(c - 2, ov, os)

            compute_chunk(c, rv, ov, os)
        return carry

    lax.fori_loop(0, NCHUNK // 2, pair_body, 0)
    wait_out(NCHUNK - 2, out_a, os_a)
    wait_out(NCHUNK - 1, out_b, os_b)


_sc_kernel = functools.partial(
    pl.kernel,
    out_type=jax.ShapeDtypeStruct((NTOK * OUT_D,), jnp.float32),
    mesh=plsc.VectorSubcoreMesh(core_axis_name="c", subcore_axis_name="s"),
    scratch_types=[
        pltpu.VMEM((TPW,), jnp.int32),
        pltpu.VMEM((TPW,), jnp.int32),
        pltpu.VMEM((M2 * R2 * Q2,), jnp.float32),
        pltpu.VMEM((CHUNK, ROW_D), jnp.float32),
        pltpu.VMEM((CHUNK, ROW_D), jnp.float32),
        pltpu.VMEM((CHUNK * OUT_D,), jnp.float32),
        pltpu.VMEM((CHUNK * OUT_D,), jnp.float32),
        pltpu.SemaphoreType.DMA,
        pltpu.SemaphoreType.DMA,
        pltpu.SemaphoreType.DMA,
        pltpu.SemaphoreType.DMA,
    ],
    compiler_params=pltpu.CompilerParams(needs_layout_passes=False),
)(_sc_body)


def kernel(x, core0, core1, core2):
    batch, sent = x.shape
    flat = x.reshape(-1).astype(jnp.int32)
    t01 = _build_t01(core0, core1)
    c2t = core2.reshape(R2, M2, Q2).transpose(1, 0, 2).reshape(M2 * R2 * Q2)
    out = _sc_kernel(flat, t01, c2t)
    return out.reshape(batch, sent, OUT_D)


# speedup vs baseline: 20.1147x; 2.3660x over previous
import functools

import jax
import jax.numpy as jnp
from jax import lax
from jax.experimental import pallas as pl
from jax.experimental.pallas import tpu as pltpu
from jax.experimental.pallas import tpu_sc as plsc

M0 = M1 = M2 = 100
Q0 = Q1 = Q2 = 4
R1 = 8
R2 = 8
NTOK = 16384 * 50
OUT_D = Q0 * Q1 * Q2
ROW_D = Q0 * Q1 * R2

NC, NS, L = 2, 16, 16
NW = NC * NS
TPW = NTOK // NW
CHUNK = 128
NCHUNK = TPW // CHUNK


def _mm_body(a_ref, b_ref, o_ref):
    o_ref[...] = jnp.dot(a_ref[...], b_ref[...],
                         preferred_element_type=jnp.float32)


def _build_m12(core1, core2):
    lhs = core1.reshape(R1 * M1 * Q1, R2)
    rhs = core2.reshape(R2, M2 * Q2)
    p = pl.pallas_call(
        _mm_body,
        out_shape=jax.ShapeDtypeStruct((R1 * M1 * Q1, M2 * Q2), jnp.float32),
    )(lhs, rhs)
    m12 = p.reshape(R1, M1, Q1, M2, Q2).transpose(1, 3, 0, 2, 4)
    return m12.reshape(M1 * M2, ROW_D)


def _sc_body(flat_hbm, m12_hbm, g0_hbm, out_hbm,
             x_v, idx_v, g0_v, rows_a, rows_b, out_a, out_b,
             gs_a, gs_b, os_a, os_b):
    wid = lax.axis_index("s") * NC + lax.axis_index("c")
    base = wid * TPW
    pltpu.sync_copy(g0_hbm, g0_v)
    pltpu.sync_copy(flat_hbm.at[pl.ds(base, TPW)], x_v)

    @plsc.parallel_loop(0, TPW // L)
    def _(t):
        xv = x_v[pl.ds(t * L, L)]
        hi = lax.div(xv, M1 * M2)
        idx_v[pl.ds(t * L, L)] = xv - hi * (M1 * M2)
        x_v[pl.ds(t * L, L)] = hi

    def start_gather(c, rows_ref, sem):
        pltpu.async_copy(
            m12_hbm.at[idx_v.at[pl.ds(c * CHUNK, CHUNK)]], rows_ref, sem)

    def compute_chunk(c, rows_ref, out_ref, osem):
        @plsc.parallel_loop(0, CHUNK // L)
        def _(tt):
            i0v = x_v[pl.ds(c * CHUNK + tt * L, L)]
            for j in range(L):
                t = tt * L + j
                gb = i0v[j] * (Q0 * R1)
                gv = (g0_v[pl.ds(gb, L)], g0_v[pl.ds(gb + L, L)])
                m = [rows_ref[t, pl.ds(r * L, L)] for r in range(R1)]
                for q0 in range(Q0):
                    k0 = q0 * R1
                    acc = gv[k0 // L][k0 % L] * m[0]
                    for r in range(1, R1):
                        k = k0 + r
                        acc = acc + gv[k // L][k % L] * m[r]
                    out_ref[pl.ds(t * OUT_D + q0 * L, L)] = acc

        pltpu.async_copy(
            out_ref,
            out_hbm.at[pl.ds((base + c * CHUNK) * OUT_D, CHUNK * OUT_D)],
            osem)

    def wait_gather(c, rows_ref, sem):
        pltpu.make_async_copy(
            m12_hbm.at[idx_v.at[pl.ds(c * CHUNK, CHUNK)]], rows_ref, sem
        ).wait()

    def wait_out(c, out_ref, osem):
        pltpu.make_async_copy(
            out_ref,
            out_hbm.at[pl.ds((base + c * CHUNK) * OUT_D, CHUNK * OUT_D)],
            osem).wait()

    start_gather(0, rows_a, gs_a)

    bufs = ((rows_a, out_a, gs_a, os_a), (rows_b, out_b, gs_b, os_b))

    def pair_body(g2, carry):
        for b in range(2):
            rv, ov, gs, os = bufs[b]
            rn, _, gn, _ = bufs[1 - b]
            c = g2 * 2 + b

            @pl.when(c + 1 < NCHUNK)
            def _():
                start_gather(c + 1, rn, gn)

            wait_gather(c, rv, gs)

            @pl.when(c >= 2)
            def _():
                wait_out(c - 2, ov, os)

            compute_chunk(c, rv, ov, os)
        return carry

    lax.fori_loop(0, NCHUNK // 2, pair_body, 0)
    wait_out(NCHUNK - 2, out_a, os_a)
    wait_out(NCHUNK - 1, out_b, os_b)


_sc_kernel = functools.partial(
    pl.kernel,
    out_type=jax.ShapeDtypeStruct((NTOK * OUT_D,), jnp.float32),
    mesh=plsc.VectorSubcoreMesh(core_axis_name="c", subcore_axis_name="s"),
    scratch_types=[
        pltpu.VMEM((TPW,), jnp.int32),
        pltpu.VMEM((TPW,), jnp.int32),
        pltpu.VMEM((M0 * Q0 * R1,), jnp.float32),
        pltpu.VMEM((CHUNK, ROW_D), jnp.float32),
        pltpu.VMEM((CHUNK, ROW_D), jnp.float32),
        pltpu.VMEM((CHUNK * OUT_D,), jnp.float32),
        pltpu.VMEM((CHUNK * OUT_D,), jnp.float32),
        pltpu.SemaphoreType.DMA,
        pltpu.SemaphoreType.DMA,
        pltpu.SemaphoreType.DMA,
        pltpu.SemaphoreType.DMA,
    ],
    compiler_params=pltpu.CompilerParams(needs_layout_passes=False),
)(_sc_body)


def kernel(x, core0, core1, core2):
    batch, sent = x.shape
    flat = x.reshape(-1).astype(jnp.int32)
    m12 = _build_m12(core1, core2)
    g0 = core0.reshape(M0 * Q0 * R1)
    out = _sc_kernel(flat, m12, g0)
    return out.reshape(batch, sent, OUT_D)
